# baseline (speedup 1.0000x reference)
import functools

import jax
import jax.numpy as jnp
import numpy as np
from jax import lax
from jax.experimental import pallas as pl
from jax.experimental.pallas import tpu as pltpu
from jax.experimental.pallas import tpu_sc as plsc

_VOCAB = 100000
_D = 64
_B = 4096
_L = 200

_NC = 2
_NS = 16
_LANES = 16
_RPT = _B // _NS
_WPT = _RPT * _L
_HALF = 50176
_VPAD = _NC * _HALF
_ZROW = _HALF
_GRP = 16

_STORED_COLS = np.array(
    [2 * k for k in range(16)] + [2 * k + 1 for k in range(16)]
    + [32 + 2 * k for k in range(16)] + [33 + 2 * k for k in range(16)],
    dtype=np.int32)
_INV_PERM = np.argsort(_STORED_COLS).astype(np.int32)


_BLK = 32


def _sc_pool(table_hbm, ids_hbm, mask_hbm, out_hbm, cnt_hbm,
             ids_v, mask_t, buf_a, buf_b, zrow_v, cnt_v, out_v,
             spmem_t, sem_a, sem_b, sem_0):
  cid = lax.axis_index("c")
  sid = lax.axis_index("s")
  base = sid * _WPT
  vbase = cid * _HALF

  @pl.when(sid == 0)
  def _stage_table():
    pltpu.async_copy(table_hbm.at[pl.ds(vbase, _HALF)],
                     spmem_t.at[pl.ds(0, _HALF)], sem_0)
    for r in range(8):
      for j in range(_D // 32):
        zrow_v[r, pl.ds(j * 32, 32)] = jnp.zeros((32,), jnp.bfloat16)
    pltpu.sync_copy(zrow_v, spmem_t.at[pl.ds(_ZROW, 8)])

  lane = lax.iota(jnp.int32, _LANES)
  hi_mask = jnp.full((_LANES,), -65536, jnp.int32)

  @pl.when(sid == 0)
  def _wait_table():
    pltpu.make_async_copy(table_hbm.at[pl.ds(vbase, _HALF)],
                          spmem_t.at[pl.ds(0, _HALF)], sem_0).wait()
  plsc.subcore_barrier()

  def issue(b, buf, sem):
    roff = b * _L
    pltpu.async_copy(spmem_t.at[ids_v.at[pl.ds(roff, 104)]],
                     buf.at[pl.ds(0, 104)], sem)
    pltpu.async_copy(spmem_t.at[ids_v.at[pl.ds(roff + 104, 96)]],
                     buf.at[pl.ds(104, 96)], sem)

  def drain(buf, sem):
    pltpu.make_async_copy(table_hbm.at[pl.ds(0, _L)], buf, sem).wait()

  def process(b, buf):
    def body(l, acc):
      v0 = plsc.bitcast(buf[l, pl.ds(0, 32)], jnp.int32)
      v1 = plsc.bitcast(buf[l, pl.ds(32, 32)], jnp.int32)
      e0 = plsc.bitcast(v0 << 16, jnp.float32)
      o0 = plsc.bitcast(v0 & hi_mask, jnp.float32)
      e1 = plsc.bitcast(v1 << 16, jnp.float32)
      o1 = plsc.bitcast(v1 & hi_mask, jnp.float32)
      return (acc[0] + e0, acc[1] + o0, acc[2] + e1, acc[3] + o1)

    acc = lax.fori_loop(
        0, _L, body,
        tuple(jnp.zeros((_LANES,), jnp.float32) for _ in range(4)))
    for c in range(4):
      out_v[b, pl.ds(c * _LANES, _LANES)] = acc[c]

  @pl.loop(0, _RPT // _BLK)
  def _block(blk):
    boff = base + blk * _BLK * _L

    pltpu.sync_copy(ids_hbm.at[pl.ds(boff, _BLK * _L)], ids_v)

    @pl.loop(0, _BLK // _GRP)
    def _prep(g):
      goff = g * _GRP * _L
      pltpu.sync_copy(mask_hbm.at[pl.ds(boff + goff, _GRP * _L)], mask_t)

      @pl.loop(0, _GRP * _L // _LANES, unroll=8)
      def _mask_ids(j):
        off = j * _LANES
        m = mask_t[pl.ds(off, _LANES)]
        loc = ids_v[pl.ds(goff + off, _LANES)] - vbase
        ok = (m > 0) & (loc >= 0) & (loc < _HALF)
        ids_v[pl.ds(goff + off, _LANES)] = jnp.where(
            ok, loc, jnp.full((_LANES,), _ZROW, jnp.int32))

      @pl.when(cid == 0)
      def _counts():
        idx0 = lane * _L

        def cbody(l, cvec):
          return cvec + plsc.load_gather(mask_t, [idx0 + l])

        cvec = lax.fori_loop(0, _L, cbody, jnp.zeros((_LANES,), jnp.int32))
        cnt_v[pl.ds(g * _GRP, _GRP)] = cvec.astype(jnp.float32)

    issue(0, buf_a, sem_a)

    @pl.loop(0, _BLK // 2)
    def _main(i):
      b0 = 2 * i
      issue(b0 + 1, buf_b, sem_b)
      drain(buf_a, sem_a)
      process(b0, buf_a)

      @pl.when(i < _BLK // 2 - 1)
      def _():
        issue(b0 + 2, buf_a, sem_a)

      drain(buf_b, sem_b)
      process(b0 + 1, buf_b)

    obase = sid * _RPT + blk * _BLK
    pltpu.sync_copy(out_v, out_hbm.at[cid, pl.ds(obase, _BLK)])

    @pl.when(cid == 0)
    def _write_counts():
      pltpu.sync_copy(cnt_v, cnt_hbm.at[pl.ds(obase, _BLK)])


@functools.cache
def _sc_pool_call():
  return pl.kernel(
      _sc_pool,
      out_type=(jax.ShapeDtypeStruct((_NC, _B, _D), jnp.float32),
                jax.ShapeDtypeStruct((_B,), jnp.float32)),
      mesh=plsc.VectorSubcoreMesh(core_axis_name="c", subcore_axis_name="s",
                                  num_cores=_NC, num_subcores=_NS),
      compiler_params=pltpu.CompilerParams(needs_layout_passes=False,
                                           use_tc_tiling_on_sc=False),
      scratch_types=[
          pltpu.VMEM((_BLK * _L,), jnp.int32),
          pltpu.VMEM((_GRP * _L,), jnp.int32),
          pltpu.VMEM((_L, _D), jnp.bfloat16),
          pltpu.VMEM((_L, _D), jnp.bfloat16),
          pltpu.VMEM((8, _D), jnp.bfloat16),
          pltpu.VMEM((_BLK,), jnp.float32),
          pltpu.VMEM((_BLK, _D), jnp.float32),
          pltpu.VMEM_SHARED((_HALF + 8, _D), jnp.bfloat16),
          pltpu.SemaphoreType.DMA,
          pltpu.SemaphoreType.DMA,
          pltpu.SemaphoreType.DMA,
      ],
  )


def _tc_finish_body(p_ref, cnt_ref, gamma_ref, beta_ref, o_ref):
  x = (p_ref[0] + p_ref[1]) / jnp.maximum(cnt_ref[...], 1.0)
  g = gamma_ref[...]
  b = beta_ref[...]
  mean = jnp.mean(x, axis=-1, keepdims=True)
  xc = x - mean
  var = jnp.mean(xc * xc, axis=-1, keepdims=True)
  y = xc * lax.rsqrt(var + 1e-5) * g + b
  nrm = jnp.sqrt(jnp.sum(y * y, axis=-1, keepdims=True))
  o_ref[...] = y / jnp.maximum(nrm, 1e-12)


@jax.jit
def kernel(table, gamma, beta, input_ids, attention_mask):
  table_b = jnp.pad(table.astype(jnp.bfloat16),
                    ((0, _VPAD - _VOCAB), (0, 0)))
  ids_flat = input_ids.astype(jnp.int32).reshape(_B * _L)
  mask_flat = attention_mask.astype(jnp.int32).reshape(_B * _L)
  sums, cnt = _sc_pool_call()(table_b, ids_flat, mask_flat)
  sums = jnp.take(sums, jnp.asarray(_INV_PERM), axis=-1)
  out = pl.pallas_call(
      _tc_finish_body,
      out_shape=jax.ShapeDtypeStruct((_B, _D), jnp.float32),
  )(sums, cnt.reshape(_B, 1), gamma.reshape(1, _D), beta.reshape(1, _D))
  return out


# speedup vs baseline: 1.0344x; 1.0344x over previous
import functools

import jax
import jax.numpy as jnp
import numpy as np
from jax import lax
from jax.experimental import pallas as pl
from jax.experimental.pallas import tpu as pltpu
from jax.experimental.pallas import tpu_sc as plsc

_VOCAB = 100000
_D = 64
_B = 4096
_L = 200

_NC = 2
_NS = 16
_LANES = 16
_RPT = _B // _NS
_WPT = _RPT * _L
_HALF = 50176
_VPAD = _NC * _HALF
_ZROW = _HALF
_GRP = 16

_STORED_COLS = np.array(
    [2 * k for k in range(16)] + [2 * k + 1 for k in range(16)]
    + [32 + 2 * k for k in range(16)] + [33 + 2 * k for k in range(16)],
    dtype=np.int32)
_INV_PERM = np.argsort(_STORED_COLS).astype(np.int32)


_BLK = 32


def _sc_pool(table_hbm, ids_hbm, mask_hbm, out_hbm, cnt_hbm,
             ids_v, mask_t, buf_a, buf_b, zrow_v, cnt_v, out_v,
             spmem_t, sem_a, sem_b, sem_0):
  cid = lax.axis_index("c")
  sid = lax.axis_index("s")
  base = sid * _WPT
  vbase = cid * _HALF

  @pl.when(sid == 0)
  def _stage_table():
    @pl.when(cid == 0)
    def _():
      pltpu.async_copy(table_hbm.at[pl.ds(0, _HALF)],
                       spmem_t.at[pl.ds(0, _HALF)], sem_0)

    @pl.when(cid == 1)
    def _():
      pltpu.async_copy(table_hbm.at[pl.ds(_HALF, _VOCAB - _HALF)],
                       spmem_t.at[pl.ds(0, _VOCAB - _HALF)], sem_0)

    for r in range(8):
      for j in range(_D // 32):
        zrow_v[r, pl.ds(j * 32, 32)] = jnp.zeros((32,), jnp.bfloat16)
    pltpu.sync_copy(zrow_v, spmem_t.at[pl.ds(_ZROW, 8)])

  lane = lax.iota(jnp.int32, _LANES)
  hi_mask = jnp.full((_LANES,), -65536, jnp.int32)

  @pl.when(sid == 0)
  def _wait_table():
    @pl.when(cid == 0)
    def _():
      pltpu.make_async_copy(table_hbm.at[pl.ds(0, _HALF)],
                            spmem_t.at[pl.ds(0, _HALF)], sem_0).wait()

    @pl.when(cid == 1)
    def _():
      pltpu.make_async_copy(table_hbm.at[pl.ds(_HALF, _VOCAB - _HALF)],
                            spmem_t.at[pl.ds(0, _VOCAB - _HALF)], sem_0).wait()
  plsc.subcore_barrier()

  def issue(b, buf, sem):
    roff = b * _L
    pltpu.async_copy(spmem_t.at[ids_v.at[pl.ds(roff, 104)]],
                     buf.at[pl.ds(0, 104)], sem)
    pltpu.async_copy(spmem_t.at[ids_v.at[pl.ds(roff + 104, 96)]],
                     buf.at[pl.ds(104, 96)], sem)

  def drain(buf, sem):
    pltpu.make_async_copy(table_hbm.at[pl.ds(0, _L)], buf, sem).wait()

  def process(b, buf):
    def body(l, acc):
      v0 = plsc.bitcast(buf[l, pl.ds(0, 32)], jnp.int32)
      v1 = plsc.bitcast(buf[l, pl.ds(32, 32)], jnp.int32)
      e0 = plsc.bitcast(v0 << 16, jnp.float32)
      o0 = plsc.bitcast(v0 & hi_mask, jnp.float32)
      e1 = plsc.bitcast(v1 << 16, jnp.float32)
      o1 = plsc.bitcast(v1 & hi_mask, jnp.float32)
      return (acc[0] + e0, acc[1] + o0, acc[2] + e1, acc[3] + o1)

    acc = lax.fori_loop(
        0, _L, body,
        tuple(jnp.zeros((_LANES,), jnp.float32) for _ in range(4)))
    for c in range(4):
      out_v[b, pl.ds(c * _LANES, _LANES)] = acc[c]

  @pl.loop(0, _RPT // _BLK)
  def _block(blk):
    boff = base + blk * _BLK * _L

    pltpu.sync_copy(ids_hbm.at[pl.ds(boff, _BLK * _L)], ids_v)

    @pl.loop(0, _BLK // _GRP)
    def _prep(g):
      goff = g * _GRP * _L
      pltpu.sync_copy(mask_hbm.at[pl.ds(boff + goff, _GRP * _L)], mask_t)

      @pl.loop(0, _GRP * _L // _LANES, unroll=8)
      def _mask_ids(j):
        off = j * _LANES
        m = mask_t[pl.ds(off, _LANES)]
        loc = ids_v[pl.ds(goff + off, _LANES)] - vbase
        ok = (m > 0) & (loc >= 0) & (loc < _HALF)
        ids_v[pl.ds(goff + off, _LANES)] = jnp.where(
            ok, loc, jnp.full((_LANES,), _ZROW, jnp.int32))

      @pl.when(cid == 0)
      def _counts():
        idx0 = lane * _L

        def cbody(l, cvec):
          return cvec + plsc.load_gather(mask_t, [idx0 + l])

        cvec = lax.fori_loop(0, _L, cbody, jnp.zeros((_LANES,), jnp.int32))
        cnt_v[pl.ds(g * _GRP, _GRP)] = cvec.astype(jnp.float32)

    issue(0, buf_a, sem_a)

    @pl.loop(0, _BLK // 2)
    def _main(i):
      b0 = 2 * i
      issue(b0 + 1, buf_b, sem_b)
      drain(buf_a, sem_a)
      process(b0, buf_a)

      @pl.when(i < _BLK // 2 - 1)
      def _():
        issue(b0 + 2, buf_a, sem_a)

      drain(buf_b, sem_b)
      process(b0 + 1, buf_b)

    obase = sid * _RPT + blk * _BLK
    pltpu.sync_copy(out_v, out_hbm.at[cid, pl.ds(obase, _BLK)])

    @pl.when(cid == 0)
    def _write_counts():
      pltpu.sync_copy(cnt_v, cnt_hbm.at[pl.ds(obase, _BLK)])


@functools.cache
def _sc_pool_call():
  return pl.kernel(
      _sc_pool,
      out_type=(jax.ShapeDtypeStruct((_NC, _B, _D), jnp.float32),
                jax.ShapeDtypeStruct((_B,), jnp.float32)),
      mesh=plsc.VectorSubcoreMesh(core_axis_name="c", subcore_axis_name="s",
                                  num_cores=_NC, num_subcores=_NS),
      compiler_params=pltpu.CompilerParams(needs_layout_passes=False,
                                           use_tc_tiling_on_sc=False),
      scratch_types=[
          pltpu.VMEM((_BLK * _L,), jnp.int32),
          pltpu.VMEM((_GRP * _L,), jnp.int32),
          pltpu.VMEM((_L, _D), jnp.bfloat16),
          pltpu.VMEM((_L, _D), jnp.bfloat16),
          pltpu.VMEM((8, _D), jnp.bfloat16),
          pltpu.VMEM((_BLK,), jnp.float32),
          pltpu.VMEM((_BLK, _D), jnp.float32),
          pltpu.VMEM_SHARED((_HALF + 8, _D), jnp.bfloat16),
          pltpu.SemaphoreType.DMA,
          pltpu.SemaphoreType.DMA,
          pltpu.SemaphoreType.DMA,
      ],
  )


def _tc_finish_body(p_ref, cnt_ref, gamma_ref, beta_ref, o_ref):
  x = (p_ref[0] + p_ref[1]) / jnp.maximum(cnt_ref[...], 1.0)
  g = gamma_ref[...]
  b = beta_ref[...]
  mean = jnp.mean(x, axis=-1, keepdims=True)
  xc = x - mean
  var = jnp.mean(xc * xc, axis=-1, keepdims=True)
  y = xc * lax.rsqrt(var + 1e-5) * g + b
  nrm = jnp.sqrt(jnp.sum(y * y, axis=-1, keepdims=True))
  o_ref[...] = y / jnp.maximum(nrm, 1e-12)


@jax.jit
def kernel(table, gamma, beta, input_ids, attention_mask):
  table_b = table.astype(jnp.bfloat16)
  ids_flat = input_ids.astype(jnp.int32).reshape(_B * _L)
  mask_flat = attention_mask.astype(jnp.int32).reshape(_B * _L)
  sums, cnt = _sc_pool_call()(table_b, ids_flat, mask_flat)
  sums = (sums.reshape(_NC, _B, 2, 2, 16).swapaxes(-1, -2)
          .reshape(_NC, _B, _D))
  out = pl.pallas_call(
      _tc_finish_body,
      out_shape=jax.ShapeDtypeStruct((_B, _D), jnp.float32),
  )(sums, cnt.reshape(_B, 1), gamma.reshape(1, _D), beta.reshape(1, _D))
  return out


# speedup vs baseline: 1.9534x; 1.8885x over previous
import functools

import jax
import jax.numpy as jnp
import numpy as np
from jax import lax
from jax.experimental import pallas as pl
from jax.experimental.pallas import tpu as pltpu
from jax.experimental.pallas import tpu_sc as plsc

_VOCAB = 100000
_D = 64
_B = 4096
_L = 200

_NC = 2
_NS = 16
_LANES = 16
_RPT = _B // _NS
_WPT = _RPT * _L
_HALF = 50176
_VPAD = _NC * _HALF
_ZROW = _HALF
_GRP = 16

_STORED_COLS = np.array(
    [2 * k for k in range(16)] + [2 * k + 1 for k in range(16)]
    + [32 + 2 * k for k in range(16)] + [33 + 2 * k for k in range(16)],
    dtype=np.int32)
_INV_PERM = np.argsort(_STORED_COLS).astype(np.int32)


_BLK = 32


def _sc_pool(table_hbm, ids_hbm, mask_hbm, out_hbm, cnt_hbm,
             ids_v, mask_t, buf_a, buf_b, zrow_v, cnt_v, out_v, nrow_s,
             spmem_t, sem_a, sem_b, sem_0):
  cid = lax.axis_index("c")
  sid = lax.axis_index("s")
  base = sid * _WPT
  vbase = cid * _HALF

  @pl.when(sid == 0)
  def _stage_table():
    @pl.when(cid == 0)
    def _():
      pltpu.async_copy(table_hbm.at[pl.ds(0, _HALF)],
                       spmem_t.at[pl.ds(0, _HALF)], sem_0)

    @pl.when(cid == 1)
    def _():
      pltpu.async_copy(table_hbm.at[pl.ds(_HALF, _VOCAB - _HALF)],
                       spmem_t.at[pl.ds(0, _VOCAB - _HALF)], sem_0)

    for r in range(8):
      for j in range(_D // 32):
        zrow_v[r, pl.ds(j * 32, 32)] = jnp.zeros((32,), jnp.bfloat16)
    pltpu.sync_copy(zrow_v, spmem_t.at[pl.ds(_ZROW, 8)])

  lane = lax.iota(jnp.int32, _LANES)
  hi_mask = jnp.full((_LANES,), -65536, jnp.int32)

  @pl.when(sid == 0)
  def _wait_table():
    @pl.when(cid == 0)
    def _():
      pltpu.make_async_copy(table_hbm.at[pl.ds(0, _HALF)],
                            spmem_t.at[pl.ds(0, _HALF)], sem_0).wait()

    @pl.when(cid == 1)
    def _():
      pltpu.make_async_copy(table_hbm.at[pl.ds(_HALF, _VOCAB - _HALF)],
                            spmem_t.at[pl.ds(0, _VOCAB - _HALF)], sem_0).wait()
  plsc.subcore_barrier()

  def issue(b, buf, sem):
    roff = b * _L
    n = nrow_s[b]
    pltpu.async_copy(spmem_t.at[ids_v.at[pl.ds(roff, 56)]],
                     buf.at[pl.ds(0, 56)], sem)

    @pl.when(n > 56)
    def _():
      pltpu.async_copy(spmem_t.at[ids_v.at[pl.ds(roff + 56, 48)]],
                       buf.at[pl.ds(56, 48)], sem)

    @pl.when(n > 104)
    def _():
      pltpu.async_copy(spmem_t.at[ids_v.at[pl.ds(roff + 104, 96)]],
                       buf.at[pl.ds(104, 96)], sem)

  def drain(b, buf, sem):
    n = nrow_s[b]
    pltpu.make_async_copy(table_hbm.at[pl.ds(0, 56)],
                          buf.at[pl.ds(0, 56)], sem).wait()

    @pl.when(n > 56)
    def _():
      pltpu.make_async_copy(table_hbm.at[pl.ds(0, 48)],
                            buf.at[pl.ds(56, 48)], sem).wait()

    @pl.when(n > 104)
    def _():
      pltpu.make_async_copy(table_hbm.at[pl.ds(0, 96)],
                            buf.at[pl.ds(104, 96)], sem).wait()

  def process(b, buf):
    def body(l, acc):
      v0 = plsc.bitcast(buf[l, pl.ds(0, 32)], jnp.int32)
      v1 = plsc.bitcast(buf[l, pl.ds(32, 32)], jnp.int32)
      e0 = plsc.bitcast(v0 << 16, jnp.float32)
      o0 = plsc.bitcast(v0 & hi_mask, jnp.float32)
      e1 = plsc.bitcast(v1 << 16, jnp.float32)
      o1 = plsc.bitcast(v1 & hi_mask, jnp.float32)
      return (acc[0] + e0, acc[1] + o0, acc[2] + e1, acc[3] + o1)

    acc = lax.fori_loop(
        0, nrow_s[b], body,
        tuple(jnp.zeros((_LANES,), jnp.float32) for _ in range(4)))
    for c in range(4):
      out_v[b, pl.ds(c * _LANES, _LANES)] = acc[c]

  @pl.loop(0, _RPT // _BLK)
  def _block(blk):
    boff = base + blk * _BLK * _L

    pltpu.sync_copy(ids_hbm.at[pl.ds(boff, _BLK * _L)],
                    ids_v.at[pl.ds(0, _BLK * _L)])

    @pl.loop(0, _BLK // _GRP)
    def _prep(g):
      goff = g * _GRP * _L
      pltpu.sync_copy(mask_hbm.at[pl.ds(boff + goff, _GRP * _L)], mask_t)

      @pl.loop(0, _GRP * _L // _LANES, unroll=8)
      def _mask_ids(j):
        off = j * _LANES
        m = mask_t[pl.ds(off, _LANES)]
        loc = ids_v[pl.ds(goff + off, _LANES)] - vbase
        ok = (m > 0) & (loc >= 0) & (loc < _HALF)
        ids_v[pl.ds(goff + off, _LANES)] = jnp.where(
            ok, loc, jnp.full((_LANES,), _ZROW, jnp.int32))

      @pl.when(cid == 0)
      def _counts():
        idx0 = lane * _L

        def cbody(l, cvec):
          return cvec + plsc.load_gather(mask_t, [idx0 + l])

        cvec = lax.fori_loop(0, _L, cbody, jnp.zeros((_LANES,), jnp.int32))
        cnt_v[pl.ds(g * _GRP, _GRP)] = cvec.astype(jnp.float32)

    @pl.loop(0, _BLK)
    def _compact(rb):
      roff = rb * _L

      def cc(j, off):
        o = j * _LANES
        idv = ids_v[pl.ds(roff + o, _LANES)]
        ok = idv < _ZROW
        ok = jnp.where(j == _L // _LANES,
                       ok & (lane < _L - (_L // _LANES) * _LANES), ok)
        plsc.store_compressed(ids_v.at[pl.ds(roff + off, _LANES)], idv, mask=ok)
        return off + plsc.all_reduce_population_count(ok)[0]

      n = lax.fori_loop(0, _L // _LANES + 1, cc, 0)
      nrow_s[rb] = n

    issue(0, buf_a, sem_a)

    @pl.loop(0, _BLK // 2)
    def _main(i):
      b0 = 2 * i
      issue(b0 + 1, buf_b, sem_b)
      drain(b0, buf_a, sem_a)
      process(b0, buf_a)

      @pl.when(i < _BLK // 2 - 1)
      def _():
        issue(b0 + 2, buf_a, sem_a)

      drain(b0 + 1, buf_b, sem_b)
      process(b0 + 1, buf_b)

    obase = sid * _RPT + blk * _BLK
    pltpu.sync_copy(out_v, out_hbm.at[cid, pl.ds(obase, _BLK)])

    @pl.when(cid == 0)
    def _write_counts():
      pltpu.sync_copy(cnt_v, cnt_hbm.at[pl.ds(obase, _BLK)])


@functools.cache
def _sc_pool_call():
  return pl.kernel(
      _sc_pool,
      out_type=(jax.ShapeDtypeStruct((_NC, _B, _D), jnp.float32),
                jax.ShapeDtypeStruct((_B,), jnp.float32)),
      mesh=plsc.VectorSubcoreMesh(core_axis_name="c", subcore_axis_name="s",
                                  num_cores=_NC, num_subcores=_NS),
      compiler_params=pltpu.CompilerParams(needs_layout_passes=False,
                                           use_tc_tiling_on_sc=False),
      scratch_types=[
          pltpu.VMEM((_BLK * _L + 16,), jnp.int32),
          pltpu.VMEM((_GRP * _L,), jnp.int32),
          pltpu.VMEM((_L, _D), jnp.bfloat16),
          pltpu.VMEM((_L, _D), jnp.bfloat16),
          pltpu.VMEM((8, _D), jnp.bfloat16),
          pltpu.VMEM((_BLK,), jnp.float32),
          pltpu.VMEM((_BLK, _D), jnp.float32),
          pltpu.SMEM((_BLK,), jnp.int32),
          pltpu.VMEM_SHARED((_HALF + 8, _D), jnp.bfloat16),
          pltpu.SemaphoreType.DMA,
          pltpu.SemaphoreType.DMA,
          pltpu.SemaphoreType.DMA,
      ],
  )


def _tc_finish_body(p_ref, cnt_ref, gamma_ref, beta_ref, o_ref):
  x = (p_ref[0] + p_ref[1]) / jnp.maximum(cnt_ref[...], 1.0)
  g = gamma_ref[...]
  b = beta_ref[...]
  mean = jnp.mean(x, axis=-1, keepdims=True)
  xc = x - mean
  var = jnp.mean(xc * xc, axis=-1, keepdims=True)
  y = xc * lax.rsqrt(var + 1e-5) * g + b
  nrm = jnp.sqrt(jnp.sum(y * y, axis=-1, keepdims=True))
  o_ref[...] = y / jnp.maximum(nrm, 1e-12)


@jax.jit
def kernel(table, gamma, beta, input_ids, attention_mask):
  table_b = table.astype(jnp.bfloat16)
  ids_flat = input_ids.astype(jnp.int32).reshape(_B * _L)
  mask_flat = attention_mask.astype(jnp.int32).reshape(_B * _L)
  sums, cnt = _sc_pool_call()(table_b, ids_flat, mask_flat)
  sums = (sums.reshape(_NC, _B, 2, 2, 16).swapaxes(-1, -2)
          .reshape(_NC, _B, _D))
  out = pl.pallas_call(
      _tc_finish_body,
      out_shape=jax.ShapeDtypeStruct((_B, _D), jnp.float32),
  )(sums, cnt.reshape(_B, 1), gamma.reshape(1, _D), beta.reshape(1, _D))
  return out


# speedup vs baseline: 2.0617x; 1.0554x over previous
import functools

import jax
import jax.numpy as jnp
import numpy as np
from jax import lax
from jax.experimental import pallas as pl
from jax.experimental.pallas import tpu as pltpu
from jax.experimental.pallas import tpu_sc as plsc

_VOCAB = 100000
_D = 64
_B = 4096
_L = 200

_NC = 2
_NS = 16
_LANES = 16
_RPT = _B // _NS
_WPT = _RPT * _L
_HALF = 50176
_VPAD = _NC * _HALF
_ZROW = _HALF
_GRP = 16

_STORED_COLS = np.array(
    [2 * k for k in range(16)] + [2 * k + 1 for k in range(16)]
    + [32 + 2 * k for k in range(16)] + [33 + 2 * k for k in range(16)],
    dtype=np.int32)
_INV_PERM = np.argsort(_STORED_COLS).astype(np.int32)


_BLK = 32


def _sc_pool(table_hbm, ids_hbm, mask_hbm, out_hbm,
             ids_v, mask_t, buf_a, buf_b, zrow_v, out_v, nrow_s,
             spmem_t, sem_a, sem_b, sem_0):
  cid = lax.axis_index("c")
  sid = lax.axis_index("s")
  base = sid * _WPT
  vbase = cid * _HALF

  @pl.when(sid == 0)
  def _stage_table():
    @pl.when(cid == 0)
    def _():
      pltpu.async_copy(table_hbm.at[pl.ds(0, _HALF)],
                       spmem_t.at[pl.ds(0, _HALF)], sem_0)

    @pl.when(cid == 1)
    def _():
      pltpu.async_copy(table_hbm.at[pl.ds(_HALF, _VOCAB - _HALF)],
                       spmem_t.at[pl.ds(0, _VOCAB - _HALF)], sem_0)

    for r in range(8):
      for j in range(_D // 32):
        zrow_v[r, pl.ds(j * 32, 32)] = jnp.zeros((32,), jnp.bfloat16)
    pltpu.sync_copy(zrow_v, spmem_t.at[pl.ds(_ZROW, 8)])

  lane = lax.iota(jnp.int32, _LANES)
  hi_mask = jnp.full((_LANES,), -65536, jnp.int32)

  @pl.when(sid == 0)
  def _wait_table():
    @pl.when(cid == 0)
    def _():
      pltpu.make_async_copy(table_hbm.at[pl.ds(0, _HALF)],
                            spmem_t.at[pl.ds(0, _HALF)], sem_0).wait()

    @pl.when(cid == 1)
    def _():
      pltpu.make_async_copy(table_hbm.at[pl.ds(_HALF, _VOCAB - _HALF)],
                            spmem_t.at[pl.ds(0, _VOCAB - _HALF)], sem_0).wait()
  plsc.subcore_barrier()

  def issue(b, buf, sem):
    roff = b * _L
    n = nrow_s[b]
    pltpu.async_copy(spmem_t.at[ids_v.at[pl.ds(roff, 56)]],
                     buf.at[pl.ds(0, 56)], sem)

    @pl.when(n > 56)
    def _():
      pltpu.async_copy(spmem_t.at[ids_v.at[pl.ds(roff + 56, 48)]],
                       buf.at[pl.ds(56, 48)], sem)

    @pl.when(n > 104)
    def _():
      pltpu.async_copy(spmem_t.at[ids_v.at[pl.ds(roff + 104, 96)]],
                       buf.at[pl.ds(104, 96)], sem)

  def drain(b, buf, sem):
    n = nrow_s[b]
    pltpu.make_async_copy(table_hbm.at[pl.ds(0, 56)],
                          buf.at[pl.ds(0, 56)], sem).wait()

    @pl.when(n > 56)
    def _():
      pltpu.make_async_copy(table_hbm.at[pl.ds(0, 48)],
                            buf.at[pl.ds(56, 48)], sem).wait()

    @pl.when(n > 104)
    def _():
      pltpu.make_async_copy(table_hbm.at[pl.ds(0, 96)],
                            buf.at[pl.ds(104, 96)], sem).wait()

  def process(b, buf):
    def body(l, acc):
      v0 = plsc.bitcast(buf[l, pl.ds(0, 32)], jnp.int32)
      v1 = plsc.bitcast(buf[l, pl.ds(32, 32)], jnp.int32)
      e0 = plsc.bitcast(v0 << 16, jnp.float32)
      o0 = plsc.bitcast(v0 & hi_mask, jnp.float32)
      e1 = plsc.bitcast(v1 << 16, jnp.float32)
      o1 = plsc.bitcast(v1 & hi_mask, jnp.float32)
      return (acc[0] + e0, acc[1] + o0, acc[2] + e1, acc[3] + o1)

    acc = lax.fori_loop(
        0, nrow_s[b], body,
        tuple(jnp.zeros((_LANES,), jnp.float32) for _ in range(4)))
    for c in range(4):
      out_v[b, pl.ds(c * _LANES, _LANES)] = acc[c]

  @pl.loop(0, _RPT // _BLK)
  def _block(blk):
    boff = base + blk * _BLK * _L

    pltpu.sync_copy(ids_hbm.at[pl.ds(boff, _BLK * _L)],
                    ids_v.at[pl.ds(0, _BLK * _L)])

    @pl.loop(0, _BLK // _GRP)
    def _prep(g):
      goff = g * _GRP * _L
      pltpu.sync_copy(mask_hbm.at[pl.ds(boff + goff, _GRP * _L)], mask_t)

      @pl.loop(0, _GRP * _L // _LANES, unroll=8)
      def _mask_ids(j):
        off = j * _LANES
        m = mask_t[pl.ds(off, _LANES)]
        loc = ids_v[pl.ds(goff + off, _LANES)] - vbase
        ok = (m > 0) & (loc >= 0) & (loc < _HALF)
        ids_v[pl.ds(goff + off, _LANES)] = jnp.where(
            ok, loc, jnp.full((_LANES,), _ZROW, jnp.int32))

    @pl.loop(0, _BLK)
    def _compact(rb):
      roff = rb * _L

      def cc(j, off):
        o = j * _LANES
        idv = ids_v[pl.ds(roff + o, _LANES)]
        ok = idv < _ZROW
        ok = jnp.where(j == _L // _LANES,
                       ok & (lane < _L - (_L // _LANES) * _LANES), ok)
        plsc.store_compressed(ids_v.at[pl.ds(roff + off, _LANES)], idv, mask=ok)
        return off + plsc.all_reduce_population_count(ok)[0]

      n = lax.fori_loop(0, _L // _LANES + 1, cc, 0)
      nrow_s[rb] = n

    issue(0, buf_a, sem_a)

    @pl.loop(0, _BLK // 2)
    def _main(i):
      b0 = 2 * i
      issue(b0 + 1, buf_b, sem_b)
      drain(b0, buf_a, sem_a)
      process(b0, buf_a)

      @pl.when(i < _BLK // 2 - 1)
      def _():
        issue(b0 + 2, buf_a, sem_a)

      drain(b0 + 1, buf_b, sem_b)
      process(b0 + 1, buf_b)

    obase = sid * _RPT + blk * _BLK
    pltpu.sync_copy(out_v, out_hbm.at[cid, pl.ds(obase, _BLK)])


@functools.cache
def _sc_pool_call():
  return pl.kernel(
      _sc_pool,
      out_type=jax.ShapeDtypeStruct((_NC, _B, _D), jnp.float32),
      mesh=plsc.VectorSubcoreMesh(core_axis_name="c", subcore_axis_name="s",
                                  num_cores=_NC, num_subcores=_NS),
      compiler_params=pltpu.CompilerParams(needs_layout_passes=False,
                                           use_tc_tiling_on_sc=False),
      scratch_types=[
          pltpu.VMEM((_BLK * _L + 16,), jnp.int32),
          pltpu.VMEM((_GRP * _L,), jnp.int32),
          pltpu.VMEM((_L, _D), jnp.bfloat16),
          pltpu.VMEM((_L, _D), jnp.bfloat16),
          pltpu.VMEM((8, _D), jnp.bfloat16),
          pltpu.VMEM((_BLK, _D), jnp.float32),
          pltpu.SMEM((_BLK,), jnp.int32),
          pltpu.VMEM_SHARED((_HALF + 8, _D), jnp.bfloat16),
          pltpu.SemaphoreType.DMA,
          pltpu.SemaphoreType.DMA,
          pltpu.SemaphoreType.DMA,
      ],
  )


def _tc_finish_body(p_ref, mask_ref, gamma_ref, beta_ref, o_ref):
  cnt = jnp.sum(mask_ref[...].astype(jnp.float32), axis=-1, keepdims=True)
  x = (p_ref[0] + p_ref[1]) / jnp.maximum(cnt, 1.0)
  g = gamma_ref[...]
  b = beta_ref[...]
  mean = jnp.mean(x, axis=-1, keepdims=True)
  xc = x - mean
  var = jnp.mean(xc * xc, axis=-1, keepdims=True)
  y = xc * lax.rsqrt(var + 1e-5) * g + b
  nrm = jnp.sqrt(jnp.sum(y * y, axis=-1, keepdims=True))
  o_ref[...] = y / jnp.maximum(nrm, 1e-12)


@jax.jit
def kernel(table, gamma, beta, input_ids, attention_mask):
  table_b = table.astype(jnp.bfloat16)
  ids_flat = input_ids.astype(jnp.int32).reshape(_B * _L)
  mask_flat = attention_mask.astype(jnp.int32).reshape(_B * _L)
  sums = _sc_pool_call()(table_b, ids_flat, mask_flat)
  sums = (sums.reshape(_NC, _B, 2, 2, 16).swapaxes(-1, -2)
          .reshape(_NC, _B, _D))
  out = pl.pallas_call(
      _tc_finish_body,
      out_shape=jax.ShapeDtypeStruct((_B, _D), jnp.float32),
  )(sums, mask_flat.reshape(_B, _L), gamma.reshape(1, _D),
    beta.reshape(1, _D))
  return out
